# baseline (speedup 1.0000x reference)
import functools

import jax
import jax.numpy as jnp
from jax import lax
from jax.experimental import pallas as pl
from jax.experimental.pallas import tpu as pltpu
from jax.experimental.pallas import tpu_sc as plsc

NUM_TOKENS = 32768
MODEL_DIM = 4096
ROUTING_DIM = 512
NUM_EXPERTS = 64
T_LAUX = 0.07

TOK_BLK = 1024
GRID = NUM_TOKENS // TOK_BLK

SC_NC = 2
SC_NS = 16
SC_NW = SC_NC * SC_NS
CHUNK = NUM_TOKENS // SC_NW
NVEC = CHUNK // 16


def _router_body(inp_ref, wredT_ref, wgT_ref, gate_ref,
                 t2w_ref, pre_ref, counts_ref, ebase_ref, laux_ref,
                 me_ref):
    b = pl.program_id(0)

    @pl.when(b == 0)
    def _init():
        counts_ref[...] = jnp.zeros_like(counts_ref)
        me_ref[...] = jnp.zeros_like(me_ref)

    x = lax.dot_general(inp_ref[...], wredT_ref[...],
                        (((1,), (0,)), ((), ())),
                        preferred_element_type=jnp.float32)
    n1 = jnp.sqrt(jnp.sum(x * x, axis=1, keepdims=True))
    xn = x / jnp.maximum(n1, 1e-4)
    wgT = wgT_ref[...]
    n2 = jnp.sqrt(jnp.sum(wgT * wgT, axis=0, keepdims=True))
    wgn = wgT / jnp.maximum(n2, 1e-4)
    logits = lax.dot_general(xn, wgn, (((1,), (0,)), ((), ())),
                             preferred_element_type=jnp.float32)

    g = gate_ref[0, 0]
    temp = jnp.where(g < 1e-4, jnp.float32(1e-4), g)
    gates1 = jax.nn.sigmoid(logits / temp)
    m = jnp.max(gates1, axis=1, keepdims=True)
    idx = lax.broadcasted_iota(jnp.int32, (TOK_BLK, NUM_EXPERTS), 1)
    e = jnp.min(jnp.where(gates1 == m, idx, NUM_EXPERTS), axis=1,
                keepdims=True)
    t2w_ref[...] = e

    hist = jnp.sum((e == idx).astype(jnp.int32), axis=0, keepdims=True)
    pre_ref[...] = counts_ref[...].reshape(1, 1, NUM_EXPERTS)
    counts_ref[...] += hist

    l2 = logits / jnp.float32(T_LAUX)
    p = jnp.exp(l2 - jnp.max(l2, axis=1, keepdims=True))
    gates = p / jnp.sum(p, axis=1, keepdims=True)
    me_ref[...] += jnp.sum(gates, axis=0, keepdims=True)

    @pl.when(b == GRID - 1)
    def _fin():
        cf = counts_ref[...].astype(jnp.float32)
        ii = lax.broadcasted_iota(jnp.int32, (NUM_EXPERTS, NUM_EXPERTS), 0)
        jj = lax.broadcasted_iota(jnp.int32, (NUM_EXPERTS, NUM_EXPERTS), 1)
        lt = (ii < jj).astype(jnp.float32)
        basef = lax.dot_general(cf, lt, (((1,), (0,)), ((), ())),
                                precision=lax.Precision.HIGHEST,
                                preferred_element_type=jnp.float32)
        ebase_ref[...] = basef.astype(jnp.int32)
        ce = cf / jnp.sum(cf) + jnp.float32(1e-6)
        laux_ref[...] = jnp.sum(me_ref[...] * ce, axis=(0, 1),
                                keepdims=True) * jnp.float32(NUM_EXPERTS)


def _router(inp, wredT, wgT, g2):
    return pl.pallas_call(
        _router_body,
        grid=(GRID,),
        in_specs=[
            pl.BlockSpec((TOK_BLK, MODEL_DIM), lambda b: (b, 0)),
            pl.BlockSpec((MODEL_DIM, ROUTING_DIM), lambda b: (0, 0)),
            pl.BlockSpec((ROUTING_DIM, NUM_EXPERTS), lambda b: (0, 0)),
            pl.BlockSpec((1, 1), lambda b: (0, 0)),
        ],
        out_specs=[
            pl.BlockSpec((TOK_BLK, 1), lambda b: (b, 0)),
            pl.BlockSpec((1, 1, NUM_EXPERTS), lambda b: (b, 0, 0)),
            pl.BlockSpec((1, NUM_EXPERTS), lambda b: (0, 0)),
            pl.BlockSpec((1, NUM_EXPERTS), lambda b: (0, 0)),
            pl.BlockSpec((1, 1), lambda b: (0, 0)),
        ],
        out_shape=[
            jax.ShapeDtypeStruct((NUM_TOKENS, 1), jnp.int32),
            jax.ShapeDtypeStruct((GRID, 1, NUM_EXPERTS), jnp.int32),
            jax.ShapeDtypeStruct((1, NUM_EXPERTS), jnp.int32),
            jax.ShapeDtypeStruct((1, NUM_EXPERTS), jnp.int32),
            jax.ShapeDtypeStruct((1, 1), jnp.float32),
        ],
        scratch_shapes=[pltpu.VMEM((1, NUM_EXPERTS), jnp.float32)],
    )(inp, wredT, wgT, g2)


def _sc_sort_body(t2w_hbm, base_hbm, out_hbm, chunk_v, basebuf_v, run_v,
                  pos_v, tok_v, sem):
    wid = lax.axis_index("s") * SC_NC + lax.axis_index("c")
    tbase = wid * CHUNK
    pltpu.sync_copy(t2w_hbm.at[pl.ds(wid * (CHUNK // 128), CHUNK // 128)],
                    chunk_v)
    pltpu.sync_copy(base_hbm.at[wid], basebuf_v)
    for k in range(NUM_EXPERTS // 16):
        run_v[pl.ds(16 * k, 16)] = basebuf_v[pl.ds(16 * k, 16)]

    zc, _ = plsc.scan_count(jnp.zeros((16,), jnp.int32))
    cbase = jnp.min(zc)

    for i in range(NVEC):
        e = chunk_v[i // 8, pl.ds(16 * (i % 8), 16)]
        cnt, last = plsc.scan_count(e)
        rank = cnt - cbase
        cur = plsc.load_gather(run_v, [e])
        r, c = divmod(i, 8)
        pos_v[r, pl.ds(16 * c, 16)] = cur + rank
        plsc.addupdate_scatter(run_v, [e], rank + 1, mask=last)
        tok_v[r, pl.ds(16 * c, 16)] = lax.iota(jnp.int32, 16) + (tbase + 16 * i)

    copies = [pltpu.async_copy(tok_v.at[j], out_hbm.at[pos_v.at[j]], sem)
              for j in range(8)]
    for cp in copies:
        cp.wait()


@functools.cache
def _make_sc_sort():
    return pl.kernel(
        _sc_sort_body,
        out_type=jax.ShapeDtypeStruct((NUM_TOKENS,), jnp.int32),
        mesh=plsc.VectorSubcoreMesh(core_axis_name="c", subcore_axis_name="s",
                                    num_cores=SC_NC, num_subcores=SC_NS),
        scratch_types=[
            pltpu.VMEM((CHUNK // 128, 128), jnp.int32),
            pltpu.VMEM((128,), jnp.int32),
            pltpu.VMEM((NUM_EXPERTS,), jnp.int32),
            pltpu.VMEM((8, 128), jnp.int32),
            pltpu.VMEM((8, 128), jnp.int32),
            pltpu.SemaphoreType.DMA,
        ],
        compiler_params=pltpu.CompilerParams(needs_layout_passes=False),
    )


def kernel(input, W_red, W_g, gating_t):
    wredT = W_red.T
    wgT = W_g.T
    g2 = gating_t.reshape(1, 1)
    t2w2d, pre3, counts, ebase, laux = _router(input, wredT, wgT, g2)
    t2w_rows = t2w2d.reshape(NUM_TOKENS // 128, 128)
    base2d = pre3.reshape(GRID, NUM_EXPERTS) + ebase
    base_pad = jnp.zeros((SC_NW, 128), jnp.int32).at[:, :NUM_EXPERTS].set(
        base2d)
    w2t = _make_sc_sort()(t2w_rows, base_pad)
    spl = counts.astype(jnp.int64)
    return (laux.reshape(()), w2t, spl, spl)


# speedup vs baseline: 1.0023x; 1.0023x over previous
import functools

import jax
import jax.numpy as jnp
from jax import lax
from jax.experimental import pallas as pl
from jax.experimental.pallas import tpu as pltpu
from jax.experimental.pallas import tpu_sc as plsc

NUM_TOKENS = 32768
MODEL_DIM = 4096
ROUTING_DIM = 512
NUM_EXPERTS = 64
T_LAUX = 0.07

TOK_BLK = 1024
GRID = NUM_TOKENS // TOK_BLK

SC_NC = 2
SC_NS = 16
SC_NW = SC_NC * SC_NS
CHUNK = NUM_TOKENS // SC_NW
NVEC = CHUNK // 16


def _router_body(inp_ref, wredT_ref, wgT_ref, gate_ref,
                 t2w_ref, pre_ref, counts_ref, ebase_ref, laux_ref,
                 me_ref):
    b = pl.program_id(0)

    @pl.when(b == 0)
    def _init():
        counts_ref[...] = jnp.zeros_like(counts_ref)
        me_ref[...] = jnp.zeros_like(me_ref)

    x = lax.dot_general(inp_ref[...], wredT_ref[...],
                        (((1,), (0,)), ((), ())),
                        preferred_element_type=jnp.float32)
    n1 = jnp.sqrt(jnp.sum(x * x, axis=1, keepdims=True))
    xn = x / jnp.maximum(n1, 1e-4)
    wgT = wgT_ref[...]
    n2 = jnp.sqrt(jnp.sum(wgT * wgT, axis=0, keepdims=True))
    wgn = wgT / jnp.maximum(n2, 1e-4)
    logits = lax.dot_general(xn, wgn, (((1,), (0,)), ((), ())),
                             preferred_element_type=jnp.float32)

    g = gate_ref[0, 0]
    temp = jnp.where(g < 1e-4, jnp.float32(1e-4), g)
    gates1 = jax.nn.sigmoid(logits / temp)
    m = jnp.max(gates1, axis=1, keepdims=True)
    idx = lax.broadcasted_iota(jnp.int32, (TOK_BLK, NUM_EXPERTS), 1)
    e = jnp.min(jnp.where(gates1 == m, idx, NUM_EXPERTS), axis=1,
                keepdims=True)
    t2w_ref[...] = e

    hist = jnp.sum((e == idx).astype(jnp.int32), axis=0, keepdims=True)
    pre_ref[...] = counts_ref[...].reshape(1, 1, NUM_EXPERTS)
    counts_ref[...] += hist

    l2 = logits / jnp.float32(T_LAUX)
    p = jnp.exp(l2 - jnp.max(l2, axis=1, keepdims=True))
    gates = p / jnp.sum(p, axis=1, keepdims=True)
    me_ref[...] += jnp.sum(gates, axis=0, keepdims=True)

    @pl.when(b == GRID - 1)
    def _fin():
        cf = counts_ref[...].astype(jnp.float32)
        ii = lax.broadcasted_iota(jnp.int32, (NUM_EXPERTS, NUM_EXPERTS), 0)
        jj = lax.broadcasted_iota(jnp.int32, (NUM_EXPERTS, NUM_EXPERTS), 1)
        lt = (ii < jj).astype(jnp.float32)
        basef = lax.dot_general(cf, lt, (((1,), (0,)), ((), ())),
                                precision=lax.Precision.HIGHEST,
                                preferred_element_type=jnp.float32)
        ebase_ref[...] = basef.astype(jnp.int32)
        ce = cf / jnp.sum(cf) + jnp.float32(1e-6)
        laux_ref[...] = jnp.sum(me_ref[...] * ce, axis=(0, 1),
                                keepdims=True) * jnp.float32(NUM_EXPERTS)


def _router(inp, wredT, wgT, g2):
    return pl.pallas_call(
        _router_body,
        grid=(GRID,),
        in_specs=[
            pl.BlockSpec((TOK_BLK, MODEL_DIM), lambda b: (b, 0)),
            pl.BlockSpec((MODEL_DIM, ROUTING_DIM), lambda b: (0, 0)),
            pl.BlockSpec((ROUTING_DIM, NUM_EXPERTS), lambda b: (0, 0)),
            pl.BlockSpec((1, 1), lambda b: (0, 0)),
        ],
        out_specs=[
            pl.BlockSpec((TOK_BLK, 1), lambda b: (b, 0)),
            pl.BlockSpec((1, 1, NUM_EXPERTS), lambda b: (b, 0, 0)),
            pl.BlockSpec((1, NUM_EXPERTS), lambda b: (0, 0)),
            pl.BlockSpec((1, NUM_EXPERTS), lambda b: (0, 0)),
            pl.BlockSpec((1, 1), lambda b: (0, 0)),
        ],
        out_shape=[
            jax.ShapeDtypeStruct((NUM_TOKENS, 1), jnp.int32),
            jax.ShapeDtypeStruct((GRID, 1, NUM_EXPERTS), jnp.int32),
            jax.ShapeDtypeStruct((1, NUM_EXPERTS), jnp.int32),
            jax.ShapeDtypeStruct((1, NUM_EXPERTS), jnp.int32),
            jax.ShapeDtypeStruct((1, 1), jnp.float32),
        ],
        scratch_shapes=[pltpu.VMEM((1, NUM_EXPERTS), jnp.float32)],
    )(inp, wredT, wgT, g2)


def _sc_sort_body(t2w_hbm, base_hbm, out_hbm, chunk_v, basebuf_v, run_v,
                  pos_v, tok_v, sem):
    wid = lax.axis_index("s") * SC_NC + lax.axis_index("c")
    tbase = wid * CHUNK
    in0 = pltpu.async_copy(
        t2w_hbm.at[pl.ds(wid * (CHUNK // 128), CHUNK // 128)], chunk_v, sem)
    in1 = pltpu.async_copy(base_hbm.at[wid], basebuf_v, sem)
    in0.wait()
    in1.wait()
    for k in range(NUM_EXPERTS // 16):
        run_v[pl.ds(16 * k, 16)] = basebuf_v[pl.ds(16 * k, 16)]

    zc, _ = plsc.scan_count(jnp.zeros((16,), jnp.int32))
    cbase = jnp.min(zc)

    copies = []
    for i in range(NVEC):
        e = chunk_v[i // 8, pl.ds(16 * (i % 8), 16)]
        cnt, last = plsc.scan_count(e)
        rank = cnt - cbase
        cur = plsc.load_gather(run_v, [e])
        r, c = divmod(i, 8)
        pos_v[r, pl.ds(16 * c, 16)] = cur + rank
        plsc.addupdate_scatter(run_v, [e], rank + 1, mask=last)
        tok_v[r, pl.ds(16 * c, 16)] = lax.iota(jnp.int32, 16) + (tbase + 16 * i)
        if i % 8 == 7:
            copies.append(
                pltpu.async_copy(tok_v.at[r], out_hbm.at[pos_v.at[r]], sem))
    for cp in copies:
        cp.wait()


@functools.cache
def _make_sc_sort():
    return pl.kernel(
        _sc_sort_body,
        out_type=jax.ShapeDtypeStruct((NUM_TOKENS,), jnp.int32),
        mesh=plsc.VectorSubcoreMesh(core_axis_name="c", subcore_axis_name="s",
                                    num_cores=SC_NC, num_subcores=SC_NS),
        scratch_types=[
            pltpu.VMEM((CHUNK // 128, 128), jnp.int32),
            pltpu.VMEM((128,), jnp.int32),
            pltpu.VMEM((NUM_EXPERTS,), jnp.int32),
            pltpu.VMEM((8, 128), jnp.int32),
            pltpu.VMEM((8, 128), jnp.int32),
            pltpu.SemaphoreType.DMA,
        ],
        compiler_params=pltpu.CompilerParams(needs_layout_passes=False),
    )


def kernel(input, W_red, W_g, gating_t):
    wredT = W_red.T
    wgT = W_g.T
    g2 = gating_t.reshape(1, 1)
    t2w2d, pre3, counts, ebase, laux = _router(input, wredT, wgT, g2)
    t2w_rows = t2w2d.reshape(NUM_TOKENS // 128, 128)
    base2d = pre3.reshape(GRID, NUM_EXPERTS) + ebase
    base_pad = jnp.zeros((SC_NW, 128), jnp.int32).at[:, :NUM_EXPERTS].set(
        base2d)
    w2t = _make_sc_sort()(t2w_rows, base_pad)
    spl = counts.astype(jnp.int64)
    return (laux.reshape(()), w2t, spl, spl)


# speedup vs baseline: 1.0294x; 1.0270x over previous
import functools

import jax
import jax.numpy as jnp
from jax import lax
from jax.experimental import pallas as pl
from jax.experimental.pallas import tpu as pltpu
from jax.experimental.pallas import tpu_sc as plsc

NUM_TOKENS = 32768
MODEL_DIM = 4096
ROUTING_DIM = 512
NUM_EXPERTS = 64
T_LAUX = 0.07

TOK_BLK = 1024
GRID = NUM_TOKENS // TOK_BLK

SC_NC = 2
SC_NS = 16
SC_NW = SC_NC * SC_NS
CHUNK = NUM_TOKENS // SC_NW
NVEC = CHUNK // 16


def _router_body(inp_ref, wred_ref, wg_ref, gate_ref,
                 t2w_ref, pre_ref, counts_ref, ebase_ref, laux_ref,
                 me_ref):
    b = pl.program_id(0)

    @pl.when(b == 0)
    def _init():
        counts_ref[...] = jnp.zeros_like(counts_ref)
        me_ref[...] = jnp.zeros_like(me_ref)

    x = lax.dot_general(inp_ref[...], wred_ref[...],
                        (((1,), (1,)), ((), ())),
                        preferred_element_type=jnp.float32)
    n1 = jnp.sqrt(jnp.sum(x * x, axis=1, keepdims=True))
    xn = x / jnp.maximum(n1, 1e-4)
    wg = wg_ref[...]
    n2 = jnp.sqrt(jnp.sum(wg * wg, axis=1, keepdims=True))
    wgn = wg / jnp.maximum(n2, 1e-4)
    logits = lax.dot_general(xn, wgn, (((1,), (1,)), ((), ())),
                             preferred_element_type=jnp.float32)

    g = gate_ref[0, 0]
    temp = jnp.where(g < 1e-4, jnp.float32(1e-4), g)
    gates1 = jax.nn.sigmoid(logits / temp)
    m = jnp.max(gates1, axis=1, keepdims=True)
    idx = lax.broadcasted_iota(jnp.int32, (TOK_BLK, NUM_EXPERTS), 1)
    e = jnp.min(jnp.where(gates1 == m, idx, NUM_EXPERTS), axis=1,
                keepdims=True)
    t2w_ref[...] = e

    hist = jnp.sum((e == idx).astype(jnp.int32), axis=0, keepdims=True)
    pre_ref[...] = counts_ref[...].reshape(1, 1, NUM_EXPERTS)
    counts_ref[...] += hist

    l2 = logits / jnp.float32(T_LAUX)
    p = jnp.exp(l2 - jnp.max(l2, axis=1, keepdims=True))
    gates = p / jnp.sum(p, axis=1, keepdims=True)
    me_ref[...] += jnp.sum(gates, axis=0, keepdims=True)

    @pl.when(b == GRID - 1)
    def _fin():
        cf = counts_ref[...].astype(jnp.float32)
        ii = lax.broadcasted_iota(jnp.int32, (NUM_EXPERTS, NUM_EXPERTS), 0)
        jj = lax.broadcasted_iota(jnp.int32, (NUM_EXPERTS, NUM_EXPERTS), 1)
        lt = (ii < jj).astype(jnp.float32)
        basef = lax.dot_general(cf, lt, (((1,), (0,)), ((), ())),
                                precision=lax.Precision.HIGHEST,
                                preferred_element_type=jnp.float32)
        ebase_ref[...] = basef.astype(jnp.int32)
        ce = cf / jnp.sum(cf) + jnp.float32(1e-6)
        laux_ref[...] = jnp.sum(me_ref[...] * ce, axis=(0, 1),
                                keepdims=True) * jnp.float32(NUM_EXPERTS)


def _router(inp, wred, wg, g2):
    return pl.pallas_call(
        _router_body,
        grid=(GRID,),
        in_specs=[
            pl.BlockSpec((TOK_BLK, MODEL_DIM), lambda b: (b, 0)),
            pl.BlockSpec((ROUTING_DIM, MODEL_DIM), lambda b: (0, 0)),
            pl.BlockSpec((NUM_EXPERTS, ROUTING_DIM), lambda b: (0, 0)),
            pl.BlockSpec((1, 1), lambda b: (0, 0)),
        ],
        out_specs=[
            pl.BlockSpec((TOK_BLK, 1), lambda b: (b, 0)),
            pl.BlockSpec((1, 1, NUM_EXPERTS), lambda b: (b, 0, 0)),
            pl.BlockSpec((1, NUM_EXPERTS), lambda b: (0, 0)),
            pl.BlockSpec((1, NUM_EXPERTS), lambda b: (0, 0)),
            pl.BlockSpec((1, 1), lambda b: (0, 0)),
        ],
        out_shape=[
            jax.ShapeDtypeStruct((NUM_TOKENS, 1), jnp.int32),
            jax.ShapeDtypeStruct((GRID, 1, NUM_EXPERTS), jnp.int32),
            jax.ShapeDtypeStruct((1, NUM_EXPERTS), jnp.int32),
            jax.ShapeDtypeStruct((1, NUM_EXPERTS), jnp.int32),
            jax.ShapeDtypeStruct((1, 1), jnp.float32),
        ],
        scratch_shapes=[pltpu.VMEM((1, NUM_EXPERTS), jnp.float32)],
    )(inp, wred, wg, g2)


def _sc_sort_body(t2w_hbm, base_hbm, out_hbm, chunk_v, basebuf_v, run_v,
                  pos_v, tok_v, sem):
    wid = lax.axis_index("s") * SC_NC + lax.axis_index("c")
    tbase = wid * CHUNK
    in0 = pltpu.async_copy(
        t2w_hbm.at[pl.ds(wid * (CHUNK // 128), CHUNK // 128)], chunk_v, sem)
    in1 = pltpu.async_copy(base_hbm.at[wid], basebuf_v, sem)
    in0.wait()
    in1.wait()
    for k in range(NUM_EXPERTS // 16):
        run_v[pl.ds(16 * k, 16)] = basebuf_v[pl.ds(16 * k, 16)]

    zc, _ = plsc.scan_count(jnp.zeros((16,), jnp.int32))
    cbase = jnp.min(zc)

    copies = []
    for i in range(NVEC):
        e = chunk_v[i // 8, pl.ds(16 * (i % 8), 16)]
        cnt, last = plsc.scan_count(e)
        rank = cnt - cbase
        cur = plsc.load_gather(run_v, [e])
        r, c = divmod(i, 8)
        pos_v[r, pl.ds(16 * c, 16)] = cur + rank
        plsc.addupdate_scatter(run_v, [e], rank + 1, mask=last)
        tok_v[r, pl.ds(16 * c, 16)] = lax.iota(jnp.int32, 16) + (tbase + 16 * i)
        if i % 8 == 7:
            copies.append(
                pltpu.async_copy(tok_v.at[r], out_hbm.at[pos_v.at[r]], sem))
    for cp in copies:
        cp.wait()


@functools.cache
def _make_sc_sort():
    return pl.kernel(
        _sc_sort_body,
        out_type=jax.ShapeDtypeStruct((NUM_TOKENS,), jnp.int32),
        mesh=plsc.VectorSubcoreMesh(core_axis_name="c", subcore_axis_name="s",
                                    num_cores=SC_NC, num_subcores=SC_NS),
        scratch_types=[
            pltpu.VMEM((CHUNK // 128, 128), jnp.int32),
            pltpu.VMEM((128,), jnp.int32),
            pltpu.VMEM((NUM_EXPERTS,), jnp.int32),
            pltpu.VMEM((8, 128), jnp.int32),
            pltpu.VMEM((8, 128), jnp.int32),
            pltpu.SemaphoreType.DMA,
        ],
        compiler_params=pltpu.CompilerParams(needs_layout_passes=False),
    )


def kernel(input, W_red, W_g, gating_t):
    g2 = gating_t.reshape(1, 1)
    t2w2d, pre3, counts, ebase, laux = _router(input, W_red, W_g, g2)
    t2w_rows = t2w2d.reshape(NUM_TOKENS // 128, 128)
    base2d = pre3.reshape(GRID, NUM_EXPERTS) + ebase
    base_pad = jnp.zeros((SC_NW, 128), jnp.int32).at[:, :NUM_EXPERTS].set(
        base2d)
    w2t = _make_sc_sort()(t2w_rows, base_pad)
    spl = counts.astype(jnp.int64)
    return (laux.reshape(()), w2t, spl, spl)


# speedup vs baseline: 1.0623x; 1.0319x over previous
import functools

import jax
import jax.numpy as jnp
from jax import lax
from jax.experimental import pallas as pl
from jax.experimental.pallas import tpu as pltpu
from jax.experimental.pallas import tpu_sc as plsc

NUM_TOKENS = 32768
MODEL_DIM = 4096
ROUTING_DIM = 512
NUM_EXPERTS = 64
T_LAUX = 0.07

TOK_BLK = 1024
GRID = NUM_TOKENS // TOK_BLK

SC_NC = 2
SC_NS = 16
SC_NW = SC_NC * SC_NS
CHUNK = NUM_TOKENS // SC_NW
NVEC = CHUNK // 16


def _router_body(inp_ref, wred_ref, wg_ref, gate_ref,
                 t2w_ref, pre_ref, counts_ref, ebase_ref, laux_ref,
                 me_ref):
    b = pl.program_id(0)

    @pl.when(b == 0)
    def _init():
        counts_ref[...] = jnp.zeros_like(counts_ref)
        me_ref[...] = jnp.zeros_like(me_ref)

    x = lax.dot_general(inp_ref[...], wred_ref[...],
                        (((1,), (1,)), ((), ())),
                        preferred_element_type=jnp.float32)
    n1 = jnp.sqrt(jnp.sum(x * x, axis=1, keepdims=True))
    xn = x / jnp.maximum(n1, 1e-4)
    wg = wg_ref[...]
    n2 = jnp.sqrt(jnp.sum(wg * wg, axis=1, keepdims=True))
    wgn = wg / jnp.maximum(n2, 1e-4)
    logits = lax.dot_general(xn, wgn, (((1,), (1,)), ((), ())),
                             preferred_element_type=jnp.float32)

    g = gate_ref[0, 0]
    temp = jnp.where(g < 1e-4, jnp.float32(1e-4), g)
    gates1 = jax.nn.sigmoid(logits / temp)
    m = jnp.max(gates1, axis=1, keepdims=True)
    idx = lax.broadcasted_iota(jnp.int32, (TOK_BLK, NUM_EXPERTS), 1)
    e = jnp.min(jnp.where(gates1 == m, idx, NUM_EXPERTS), axis=1,
                keepdims=True)
    t2w_ref[...] = e.reshape(1, TOK_BLK // 128, 128)

    hist = jnp.sum((e == idx).astype(jnp.int32), axis=0, keepdims=True)
    pre_ref[...] = counts_ref[...].reshape(1, 1, NUM_EXPERTS)
    counts_ref[...] += hist

    l2 = logits / jnp.float32(T_LAUX)
    p = jnp.exp(l2 - jnp.max(l2, axis=1, keepdims=True))
    gates = p / jnp.sum(p, axis=1, keepdims=True)
    me_ref[...] += jnp.sum(gates, axis=0, keepdims=True)

    @pl.when(b == GRID - 1)
    def _fin():
        cf = counts_ref[...].astype(jnp.float32)
        ii = lax.broadcasted_iota(jnp.int32, (NUM_EXPERTS, NUM_EXPERTS), 0)
        jj = lax.broadcasted_iota(jnp.int32, (NUM_EXPERTS, NUM_EXPERTS), 1)
        lt = (ii < jj).astype(jnp.float32)
        basef = lax.dot_general(cf, lt, (((1,), (0,)), ((), ())),
                                precision=lax.Precision.HIGHEST,
                                preferred_element_type=jnp.float32)
        ebase_ref[...] = basef.astype(jnp.int32)
        ce = cf / jnp.sum(cf) + jnp.float32(1e-6)
        laux_ref[...] = jnp.sum(me_ref[...] * ce, axis=(0, 1),
                                keepdims=True) * jnp.float32(NUM_EXPERTS)


def _router(inp, wred, wg, g2):
    return pl.pallas_call(
        _router_body,
        grid=(GRID,),
        in_specs=[
            pl.BlockSpec((TOK_BLK, MODEL_DIM), lambda b: (b, 0)),
            pl.BlockSpec((ROUTING_DIM, MODEL_DIM), lambda b: (0, 0)),
            pl.BlockSpec((NUM_EXPERTS, ROUTING_DIM), lambda b: (0, 0)),
            pl.BlockSpec((1, 1), lambda b: (0, 0)),
        ],
        out_specs=[
            pl.BlockSpec((1, TOK_BLK // 128, 128), lambda b: (b, 0, 0)),
            pl.BlockSpec((1, 1, NUM_EXPERTS), lambda b: (b, 0, 0)),
            pl.BlockSpec((1, NUM_EXPERTS), lambda b: (0, 0)),
            pl.BlockSpec((1, NUM_EXPERTS), lambda b: (0, 0)),
            pl.BlockSpec((1, 1), lambda b: (0, 0)),
        ],
        out_shape=[
            jax.ShapeDtypeStruct((GRID, TOK_BLK // 128, 128), jnp.int32),
            jax.ShapeDtypeStruct((GRID, 1, NUM_EXPERTS), jnp.int32),
            jax.ShapeDtypeStruct((1, NUM_EXPERTS), jnp.int32),
            jax.ShapeDtypeStruct((1, NUM_EXPERTS), jnp.int32),
            jax.ShapeDtypeStruct((1, 1), jnp.float32),
        ],
        scratch_shapes=[pltpu.VMEM((1, NUM_EXPERTS), jnp.float32)],
    )(inp, wred, wg, g2)


def _sc_sort_body(t2w_hbm, base_hbm, out_hbm, chunk_v, basebuf_v, run_v,
                  pos_v, tok_v, sem):
    wid = lax.axis_index("s") * SC_NC + lax.axis_index("c")
    tbase = wid * CHUNK
    in0 = pltpu.async_copy(
        t2w_hbm.at[pl.ds(wid * (CHUNK // 128), CHUNK // 128)], chunk_v, sem)
    in1 = pltpu.async_copy(base_hbm.at[wid], basebuf_v, sem)
    in0.wait()
    in1.wait()
    for k in range(NUM_EXPERTS // 16):
        run_v[pl.ds(16 * k, 16)] = basebuf_v[pl.ds(16 * k, 16)]

    zc, _ = plsc.scan_count(jnp.zeros((16,), jnp.int32))
    cbase = jnp.min(zc)

    copies = []
    for i in range(NVEC):
        e = chunk_v[i // 8, pl.ds(16 * (i % 8), 16)]
        cnt, last = plsc.scan_count(e)
        rank = cnt - cbase
        cur = plsc.load_gather(run_v, [e])
        r, c = divmod(i, 8)
        pos_v[r, pl.ds(16 * c, 16)] = cur + rank
        plsc.addupdate_scatter(run_v, [e], rank + 1, mask=last)
        tok_v[r, pl.ds(16 * c, 16)] = lax.iota(jnp.int32, 16) + (tbase + 16 * i)
        if i % 8 == 7:
            copies.append(
                pltpu.async_copy(tok_v.at[r], out_hbm.at[pos_v.at[r]], sem))
    for cp in copies:
        cp.wait()


@functools.cache
def _make_sc_sort():
    return pl.kernel(
        _sc_sort_body,
        out_type=jax.ShapeDtypeStruct((NUM_TOKENS,), jnp.int32),
        mesh=plsc.VectorSubcoreMesh(core_axis_name="c", subcore_axis_name="s",
                                    num_cores=SC_NC, num_subcores=SC_NS),
        scratch_types=[
            pltpu.VMEM((CHUNK // 128, 128), jnp.int32),
            pltpu.VMEM((128,), jnp.int32),
            pltpu.VMEM((NUM_EXPERTS,), jnp.int32),
            pltpu.VMEM((8, 128), jnp.int32),
            pltpu.VMEM((8, 128), jnp.int32),
            pltpu.SemaphoreType.DMA,
        ],
        compiler_params=pltpu.CompilerParams(needs_layout_passes=False),
    )


def kernel(input, W_red, W_g, gating_t):
    g2 = gating_t.reshape(1, 1)
    t2w2d, pre3, counts, ebase, laux = _router(input, W_red, W_g, g2)
    t2w_rows = t2w2d.reshape(NUM_TOKENS // 128, 128)
    base2d = pre3.reshape(GRID, NUM_EXPERTS) + ebase
    base_pad = jnp.zeros((SC_NW, 128), jnp.int32).at[:, :NUM_EXPERTS].set(
        base2d)
    w2t = _make_sc_sort()(t2w_rows, base_pad)
    spl = counts.astype(jnp.int64)
    return (laux.reshape(()), w2t, spl, spl)
